# Initial kernel scaffold; baseline (speedup 1.0000x reference)
import jax
import jax.numpy as jnp
from jax.experimental import pallas as pl


def kernel(user, pos_item, neg_item, user_table, item_table, W1, b1, W2, b2):
    raise NotImplementedError("write your pallas kernel here")


# speedup vs baseline: 23.1216x; 23.1216x over previous
import functools

import jax
import jax.numpy as jnp
from jax import lax
from jax.experimental import pallas as pl
from jax.experimental.pallas import tpu as pltpu
from jax.experimental.pallas import tpu_sc as plsc

B = 16384
D = 256
NC = 2
NS = 16
NW = NC * NS
ROWS_PER_W = B // NW
CHUNK = 128
NCHUNK = ROWS_PER_W // CHUNK


def _sc_gather_body(user_hbm, pos_hbm, neg_hbm, utab_hbm, itab_hbm,
                    ue_hbm, pe_hbm, ne_hbm, idx_v, rows_v, sem):
    wid = lax.axis_index("s") * NC + lax.axis_index("c")
    base = wid * ROWS_PER_W
    for idx_hbm, tab_hbm, out_hbm in (
        (user_hbm, utab_hbm, ue_hbm),
        (pos_hbm, itab_hbm, pe_hbm),
        (neg_hbm, itab_hbm, ne_hbm),
    ):
        for c in range(NCHUNK):
            off = base + c * CHUNK
            pltpu.sync_copy(idx_hbm.at[pl.ds(off, CHUNK)], idx_v)
            pltpu.async_copy(tab_hbm.at[idx_v], rows_v, sem).wait()
            pltpu.sync_copy(rows_v, out_hbm.at[pl.ds(off, CHUNK)])


_sc_gather = functools.partial(
    pl.kernel,
    mesh=plsc.VectorSubcoreMesh(core_axis_name="c", subcore_axis_name="s",
                                num_cores=NC, num_subcores=NS),
    out_type=(
        jax.ShapeDtypeStruct((B, D), jnp.float32),
        jax.ShapeDtypeStruct((B, D), jnp.float32),
        jax.ShapeDtypeStruct((B, D), jnp.float32),
    ),
    scratch_types=[
        pltpu.VMEM((CHUNK,), jnp.int32),
        pltpu.VMEM((CHUNK, D), jnp.float32),
        pltpu.SemaphoreType.DMA,
    ],
)(_sc_gather_body)


def _tc_body(ue_ref, pe_ref, ne_ref, w1_ref, b1_ref, w2_ref, b2_ref,
             pos_ref, neg_ref):
    m = 0.5 * (ue_ref[...] + pe_ref[...])
    w1 = w1_ref[...]
    w2 = w2_ref[...]
    u1 = jnp.dot(m, w1, preferred_element_type=jnp.float32) + b1_ref[...]
    n1 = jnp.dot(ne_ref[...], w1, preferred_element_type=jnp.float32) + b1_ref[...]
    u2 = jnp.dot(jnp.maximum(u1, 0.0), w2,
                 preferred_element_type=jnp.float32) + b2_ref[...]
    n2 = jnp.dot(jnp.maximum(n1, 0.0), w2,
                 preferred_element_type=jnp.float32) + b2_ref[...]
    pos_ref[...] = jnp.sum(u2 * u2, axis=1, keepdims=True)
    neg_ref[...] = jnp.sum(u2 * n2, axis=1, keepdims=True)


BLK = 1024


def _tc_compute(ue, pe, ne, W1, b1, W2, b2):
    grid = (B // BLK,)
    row_spec = pl.BlockSpec((BLK, D), lambda i: (i, 0))
    w_spec = pl.BlockSpec((D, D), lambda i: (0, 0))
    b_spec = pl.BlockSpec((1, D), lambda i: (0, 0))
    out_spec = pl.BlockSpec((BLK, 1), lambda i: (i, 0))
    return pl.pallas_call(
        _tc_body,
        grid=grid,
        in_specs=[row_spec, row_spec, row_spec, w_spec, b_spec, w_spec, b_spec],
        out_specs=[out_spec, out_spec],
        out_shape=[
            jax.ShapeDtypeStruct((B, 1), jnp.float32),
            jax.ShapeDtypeStruct((B, 1), jnp.float32),
        ],
    )(ue, pe, ne, W1, b1, W2, b2)


def kernel(user, pos_item, neg_item, user_table, item_table, W1, b1, W2, b2):
    user = user.astype(jnp.int32)
    pos_item = pos_item.astype(jnp.int32)
    neg_item = neg_item.astype(jnp.int32)
    ue, pe, ne = _sc_gather(user, pos_item, neg_item, user_table, item_table)
    pos, neg = _tc_compute(ue, pe, ne, W1, b1.reshape(1, D), W2,
                           b2.reshape(1, D))
    return (pos, neg)


# speedup vs baseline: 26.9766x; 1.1667x over previous
import functools

import jax
import jax.numpy as jnp
from jax import lax
from jax.experimental import pallas as pl
from jax.experimental.pallas import tpu as pltpu
from jax.experimental.pallas import tpu_sc as plsc

B = 16384
D = 256
LANES = 16
NC = 2
NS = 16
NW = NC * NS
ROWS_PER_W = B // NW
CHUNK = 64
NCH = ROWS_PER_W // CHUNK


def _add_into(dst, src):
    def row(r, _):
        for j in range(D // LANES):
            sl = pl.ds(j * LANES, LANES)
            dst[r, sl] = dst[r, sl] + src[r, sl]
        return 0
    lax.fori_loop(0, CHUNK, row, 0, unroll=False)


def _sc_gather_body(user_hbm, pos_hbm, neg_hbm, utab_hbm, itab_hbm,
                    s_hbm, ne_hbm,
                    iu, ip, inn, ub0, pb0, nb0, ub1, pb1, nb1,
                    gsem0, gsem1, wsem):
    wid = lax.axis_index("s") * NC + lax.axis_index("c")
    base = wid * ROWS_PER_W
    pltpu.sync_copy(user_hbm.at[pl.ds(base, ROWS_PER_W)], iu)
    pltpu.sync_copy(pos_hbm.at[pl.ds(base, ROWS_PER_W)], ip)
    pltpu.sync_copy(neg_hbm.at[pl.ds(base, ROWS_PER_W)], inn)

    ub = (ub0, ub1)
    pb = (pb0, pb1)
    nb = (nb0, nb1)
    gsem = (gsem0, gsem1)

    def issue_gathers(c):
        s = c % 2
        isl = pl.ds(c * CHUNK, CHUNK)
        return (
            pltpu.async_copy(utab_hbm.at[iu.at[isl]], ub[s], gsem[s]),
            pltpu.async_copy(itab_hbm.at[ip.at[isl]], pb[s], gsem[s]),
            pltpu.async_copy(itab_hbm.at[inn.at[isl]], nb[s], gsem[s]),
        )

    def issue_writebacks(c):
        s = c % 2
        osl = pl.ds(base + c * CHUNK, CHUNK)
        return (
            pltpu.async_copy(ub[s], s_hbm.at[osl], wsem),
            pltpu.async_copy(nb[s], ne_hbm.at[osl], wsem),
        )

    g = issue_gathers(0)
    wb = None
    for c in range(NCH):
        if c + 1 < NCH:
            if wb is not None:
                wb[0].wait()
                wb[1].wait()
                wb = None
            g_next = issue_gathers(c + 1)
        else:
            g_next = None
        g[0].wait()
        g[1].wait()
        g[2].wait()
        s = c % 2
        _add_into(ub[s], pb[s])
        if wb is not None:
            wb[0].wait()
            wb[1].wait()
        wb = issue_writebacks(c)
        g = g_next
    wb[0].wait()
    wb[1].wait()


_sc_gather = functools.partial(
    pl.kernel,
    mesh=plsc.VectorSubcoreMesh(core_axis_name="c", subcore_axis_name="s",
                                num_cores=NC, num_subcores=NS),
    out_type=(
        jax.ShapeDtypeStruct((B, D), jnp.float32),
        jax.ShapeDtypeStruct((B, D), jnp.float32),
    ),
    scratch_types=[
        pltpu.VMEM((ROWS_PER_W,), jnp.int32),
        pltpu.VMEM((ROWS_PER_W,), jnp.int32),
        pltpu.VMEM((ROWS_PER_W,), jnp.int32),
        pltpu.VMEM((CHUNK, D), jnp.float32),
        pltpu.VMEM((CHUNK, D), jnp.float32),
        pltpu.VMEM((CHUNK, D), jnp.float32),
        pltpu.VMEM((CHUNK, D), jnp.float32),
        pltpu.VMEM((CHUNK, D), jnp.float32),
        pltpu.VMEM((CHUNK, D), jnp.float32),
        pltpu.SemaphoreType.DMA,
        pltpu.SemaphoreType.DMA,
        pltpu.SemaphoreType.DMA,
    ],
)(_sc_gather_body)


def _tc_body(s_ref, ne_ref, w1_ref, b1_ref, w2_ref, b2_ref,
             pos_ref, neg_ref):
    w1 = w1_ref[...]
    w2 = w2_ref[...]
    u1 = 0.5 * jnp.dot(s_ref[...], w1,
                       preferred_element_type=jnp.float32) + b1_ref[...]
    n1 = jnp.dot(ne_ref[...], w1, preferred_element_type=jnp.float32) + b1_ref[...]
    u2 = jnp.dot(jnp.maximum(u1, 0.0), w2,
                 preferred_element_type=jnp.float32) + b2_ref[...]
    n2 = jnp.dot(jnp.maximum(n1, 0.0), w2,
                 preferred_element_type=jnp.float32) + b2_ref[...]
    pos_ref[...] = jnp.sum(u2 * u2, axis=1, keepdims=True)
    neg_ref[...] = jnp.sum(u2 * n2, axis=1, keepdims=True)


BLK = 1024


def _tc_compute(s, ne, W1, b1, W2, b2):
    grid = (B // BLK,)
    row_spec = pl.BlockSpec((BLK, D), lambda i: (i, 0))
    w_spec = pl.BlockSpec((D, D), lambda i: (0, 0))
    b_spec = pl.BlockSpec((1, D), lambda i: (0, 0))
    out_spec = pl.BlockSpec((BLK, 1), lambda i: (i, 0))
    return pl.pallas_call(
        _tc_body,
        grid=grid,
        in_specs=[row_spec, row_spec, w_spec, b_spec, w_spec, b_spec],
        out_specs=[out_spec, out_spec],
        out_shape=[
            jax.ShapeDtypeStruct((B, 1), jnp.float32),
            jax.ShapeDtypeStruct((B, 1), jnp.float32),
        ],
    )(s, ne, W1, b1, W2, b2)


def kernel(user, pos_item, neg_item, user_table, item_table, W1, b1, W2, b2):
    user = user.astype(jnp.int32)
    pos_item = pos_item.astype(jnp.int32)
    neg_item = neg_item.astype(jnp.int32)
    s, ne = _sc_gather(user, pos_item, neg_item, user_table, item_table)
    pos, neg = _tc_compute(s, ne, W1, b1.reshape(1, D), W2, b2.reshape(1, D))
    return (pos, neg)
